# baseline (device time: 30254 ns/iter reference)
import jax
import jax.numpy as jnp
from jax import lax
from jax.experimental import pallas as pl
from jax.experimental.pallas import tpu as pltpu

N_RING = 4
CHUNK = 128
WIRE_DTYPE = jnp.bfloat16


def kernel(x, dy):
    k_per, m = x.shape
    _, n = dy.shape

    def body(
        x_ref,
        dy_ref,
        out_ref,
        partial_ref,
        rrecv,
        lrecv,
        rssem,
        rrsem,
        lssem,
        lrsem,
    ):
        xi = lax.axis_index("x")
        yi = lax.axis_index("y")
        zi = lax.axis_index("z")
        right = (zi + 1) % N_RING
        left = (zi - 1) % N_RING

        barrier_sem = pltpu.get_barrier_semaphore()
        for nbr in (left, right):
            pl.semaphore_signal(
                barrier_sem,
                inc=1,
                device_id=(xi, yi, nbr),
                device_id_type=pl.DeviceIdType.MESH,
            )
        pl.semaphore_wait(barrier_sem, 2)

        def compute_chunk(c):
            partial_ref[pl.ds(c * CHUNK, CHUNK), :] = lax.dot_general(
                x_ref[:, pl.ds(c * CHUNK, CHUNK)],
                dy_ref[...],
                dimension_numbers=(((0,), (0,)), ((), ())),
                preferred_element_type=jnp.float32,
            )

        def rfwd(c):
            return pltpu.make_async_remote_copy(
                src_ref=rrecv.at[c],
                dst_ref=rrecv.at[c],
                send_sem=rssem.at[c],
                recv_sem=rrsem.at[c],
                device_id=(xi, yi, right),
                device_id_type=pl.DeviceIdType.MESH,
            )

        def lfwd(c):
            return pltpu.make_async_remote_copy(
                src_ref=lrecv.at[c],
                dst_ref=lrecv.at[c],
                send_sem=lssem.at[c],
                recv_sem=lrsem.at[c],
                device_id=(xi, yi, left),
                device_id_type=pl.DeviceIdType.MESH,
            )

        @pl.when(zi <= 1)
        def _():
            for c in (3, 2, 1):
                compute_chunk(c)

                @pl.when(zi == 0)
                def _(c=c):
                    rrecv[c, :, :] = partial_ref[
                        pl.ds(c * CHUNK, CHUNK), :
                    ].astype(WIRE_DTYPE)
                    rfwd(c).start()

            compute_chunk(0)

        @pl.when(zi >= 2)
        def _():
            for c in (0, 1, 2):
                compute_chunk(c)

                @pl.when(zi == 3)
                def _(c=c):
                    lrecv[c, :, :] = partial_ref[
                        pl.ds(c * CHUNK, CHUNK), :
                    ].astype(WIRE_DTYPE)
                    lfwd(c).start()

            compute_chunk(3)

        for s in (1, 2, 3):
            for rc in (1, 2, 3):
                z_m = rc + s - 3
                if 1 <= z_m <= rc:

                    @pl.when(zi == z_m)
                    def _(rc=rc, z_m=z_m):
                        rfwd(rc).wait_recv()
                        if rc > z_m:
                            rrecv[rc, :, :] = (
                                rrecv[rc, :, :].astype(jnp.float32)
                                + partial_ref[pl.ds(rc * CHUNK, CHUNK), :]
                            ).astype(WIRE_DTYPE)
                            rfwd(rc).start()

            for lc in (0, 1, 2):
                z_m = lc + 3 - s
                if lc <= z_m <= 2:

                    @pl.when(zi == z_m)
                    def _(lc=lc, z_m=z_m):
                        lfwd(lc).wait_recv()
                        if lc < z_m:
                            lrecv[lc, :, :] = (
                                lrecv[lc, :, :].astype(jnp.float32)
                                + partial_ref[pl.ds(lc * CHUNK, CHUNK), :]
                            ).astype(WIRE_DTYPE)
                            lfwd(lc).start()

        for z_s in range(N_RING):

            @pl.when(zi == z_s)
            def _(z_s=z_s):
                val = partial_ref[pl.ds(z_s * CHUNK, CHUNK), :]
                if z_s >= 1:
                    val = val + rrecv[z_s, :, :].astype(jnp.float32)
                if z_s <= 2:
                    val = val + lrecv[z_s, :, :].astype(jnp.float32)
                out_ref[...] = val

        for z_s in range(N_RING):

            @pl.when(zi == z_s)
            def _(z_s=z_s):
                for rc in range(z_s + 1, N_RING):
                    rfwd(rc).wait_send()
                for lc in range(0, z_s):
                    lfwd(lc).wait_send()

    return pl.pallas_call(
        body,
        out_shape=jax.ShapeDtypeStruct((CHUNK, n), jnp.float32),
        in_specs=[
            pl.BlockSpec(memory_space=pltpu.VMEM),
            pl.BlockSpec(memory_space=pltpu.VMEM),
        ],
        out_specs=pl.BlockSpec(memory_space=pltpu.VMEM),
        scratch_shapes=[
            pltpu.VMEM((m, n), jnp.float32),
            pltpu.VMEM((N_RING, CHUNK, n), WIRE_DTYPE),
            pltpu.VMEM((N_RING, CHUNK, n), WIRE_DTYPE),
            pltpu.SemaphoreType.DMA((N_RING,)),
            pltpu.SemaphoreType.DMA((N_RING,)),
            pltpu.SemaphoreType.DMA((N_RING,)),
            pltpu.SemaphoreType.DMA((N_RING,)),
        ],
        compiler_params=pltpu.CompilerParams(collective_id=0),
    )(x, dy)


# device time: 26907 ns/iter; 1.1244x vs baseline; 1.1244x over previous
import jax
import jax.numpy as jnp
from jax import lax
from jax.experimental import pallas as pl
from jax.experimental.pallas import tpu as pltpu

N_RING = 4
CHUNK = 128
WIRE_DTYPE = jnp.bfloat16


def kernel(x, dy):
    k_per, m = x.shape
    _, n = dy.shape
    nh = n // 2

    def body(
        x_ref,
        dy_ref,
        out_ref,
        partial_ref,
        rrecv,
        lrecv,
        xsend,
        xrecv,
        rssem,
        rrsem,
        lssem,
        lrsem,
        xssem,
        xrsem,
    ):
        xi = lax.axis_index("x")
        yi = lax.axis_index("y")
        zi = lax.axis_index("z")
        right = (zi + 1) % N_RING
        left = (zi - 1) % N_RING

        barrier_sem = pltpu.get_barrier_semaphore()
        for dev in ((xi, yi, left), (xi, yi, right), (1 - xi, yi, zi)):
            pl.semaphore_signal(
                barrier_sem,
                inc=1,
                device_id=dev,
                device_id_type=pl.DeviceIdType.MESH,
            )
        pl.semaphore_wait(barrier_sem, 3)

        def compute_chunk(c, xh):
            partial_ref[pl.ds(c * CHUNK, CHUNK), :] = lax.dot_general(
                x_ref[:, pl.ds(c * CHUNK, CHUNK)],
                dy_ref[:, pl.ds(xh * nh, nh)],
                dimension_numbers=(((0,), (0,)), ((), ())),
                preferred_element_type=jnp.float32,
            )

        def rfwd(c):
            return pltpu.make_async_remote_copy(
                src_ref=rrecv.at[c],
                dst_ref=rrecv.at[c],
                send_sem=rssem.at[c],
                recv_sem=rrsem.at[c],
                device_id=(xi, yi, right),
                device_id_type=pl.DeviceIdType.MESH,
            )

        def lfwd(c):
            return pltpu.make_async_remote_copy(
                src_ref=lrecv.at[c],
                dst_ref=lrecv.at[c],
                send_sem=lssem.at[c],
                recv_sem=lrsem.at[c],
                device_id=(xi, yi, left),
                device_id_type=pl.DeviceIdType.MESH,
            )

        def xswap():
            return pltpu.make_async_remote_copy(
                src_ref=xsend,
                dst_ref=xrecv,
                send_sem=xssem,
                recv_sem=xrsem,
                device_id=(1 - xi, yi, zi),
                device_id_type=pl.DeviceIdType.MESH,
            )

        for xh in (0, 1):

            @pl.when((xi == xh) & (zi <= 1))
            def _(xh=xh):
                for c in (3, 2, 1):
                    compute_chunk(c, xh)

                    @pl.when(zi == 0)
                    def _(c=c):
                        rrecv[c, :, :] = partial_ref[
                            pl.ds(c * CHUNK, CHUNK), :
                        ].astype(WIRE_DTYPE)
                        rfwd(c).start()

                compute_chunk(0, xh)

            @pl.when((xi == xh) & (zi >= 2))
            def _(xh=xh):
                for c in (0, 1, 2):
                    compute_chunk(c, xh)

                    @pl.when(zi == 3)
                    def _(c=c):
                        lrecv[c, :, :] = partial_ref[
                            pl.ds(c * CHUNK, CHUNK), :
                        ].astype(WIRE_DTYPE)
                        lfwd(c).start()

                compute_chunk(3, xh)

        for s in (1, 2, 3):
            for rc in (1, 2, 3):
                z_m = rc + s - 3
                if 1 <= z_m <= rc:

                    @pl.when(zi == z_m)
                    def _(rc=rc, z_m=z_m):
                        rfwd(rc).wait_recv()
                        if rc > z_m:
                            rrecv[rc, :, :] = (
                                rrecv[rc, :, :].astype(jnp.float32)
                                + partial_ref[pl.ds(rc * CHUNK, CHUNK), :]
                            ).astype(WIRE_DTYPE)
                            rfwd(rc).start()

            for lc in (0, 1, 2):
                z_m = lc + 3 - s
                if lc <= z_m <= 2:

                    @pl.when(zi == z_m)
                    def _(lc=lc, z_m=z_m):
                        lfwd(lc).wait_recv()
                        if lc < z_m:
                            lrecv[lc, :, :] = (
                                lrecv[lc, :, :].astype(jnp.float32)
                                + partial_ref[pl.ds(lc * CHUNK, CHUNK), :]
                            ).astype(WIRE_DTYPE)
                            lfwd(lc).start()

        for z_s in range(N_RING):

            @pl.when(zi == z_s)
            def _(z_s=z_s):
                val = partial_ref[pl.ds(z_s * CHUNK, CHUNK), :]
                if z_s >= 1:
                    val = val + rrecv[z_s, :, :].astype(jnp.float32)
                if z_s <= 2:
                    val = val + lrecv[z_s, :, :].astype(jnp.float32)
                xsend[...] = val.astype(WIRE_DTYPE)
                for xh in (0, 1):

                    @pl.when(xi == xh)
                    def _(xh=xh):
                        out_ref[:, pl.ds(xh * nh, nh)] = val

        swap = xswap()
        swap.start()
        swap.wait_recv()
        for xh in (0, 1):

            @pl.when(xi == xh)
            def _(xh=xh):
                out_ref[:, pl.ds((1 - xh) * nh, nh)] = xrecv[...].astype(
                    jnp.float32
                )

        swap.wait_send()
        for z_s in range(N_RING):

            @pl.when(zi == z_s)
            def _(z_s=z_s):
                for rc in range(z_s + 1, N_RING):
                    rfwd(rc).wait_send()
                for lc in range(0, z_s):
                    lfwd(lc).wait_send()

    return pl.pallas_call(
        body,
        out_shape=jax.ShapeDtypeStruct((CHUNK, n), jnp.float32),
        in_specs=[
            pl.BlockSpec(memory_space=pltpu.VMEM),
            pl.BlockSpec(memory_space=pltpu.VMEM),
        ],
        out_specs=pl.BlockSpec(memory_space=pltpu.VMEM),
        scratch_shapes=[
            pltpu.VMEM((m, nh), jnp.float32),
            pltpu.VMEM((N_RING, CHUNK, nh), WIRE_DTYPE),
            pltpu.VMEM((N_RING, CHUNK, nh), WIRE_DTYPE),
            pltpu.VMEM((CHUNK, nh), WIRE_DTYPE),
            pltpu.VMEM((CHUNK, nh), WIRE_DTYPE),
            pltpu.SemaphoreType.DMA((N_RING,)),
            pltpu.SemaphoreType.DMA((N_RING,)),
            pltpu.SemaphoreType.DMA((N_RING,)),
            pltpu.SemaphoreType.DMA((N_RING,)),
            pltpu.SemaphoreType.DMA,
            pltpu.SemaphoreType.DMA,
        ],
        compiler_params=pltpu.CompilerParams(collective_id=0),
    )(x, dy)


# device time: 25249 ns/iter; 1.1982x vs baseline; 1.0657x over previous
import jax
import jax.numpy as jnp
from jax import lax
from jax.experimental import pallas as pl
from jax.experimental.pallas import tpu as pltpu

N_RING = 4
CHUNK = 128
NBLK = 2
BLK = CHUNK // NBLK
WIRE_DTYPE = jnp.bfloat16


def kernel(x, dy):
    k_per, m = x.shape
    _, n = dy.shape
    nh = n // 2

    def body(
        x_ref,
        dy_ref,
        out_ref,
        partial_ref,
        rrecv,
        lrecv,
        xsend,
        xrecv,
        rssem,
        rrsem,
        lssem,
        lrsem,
        xssem,
        xrsem,
    ):
        xi = lax.axis_index("x")
        yi = lax.axis_index("y")
        zi = lax.axis_index("z")
        right = (zi + 1) % N_RING
        left = (zi - 1) % N_RING

        barrier_sem = pltpu.get_barrier_semaphore()
        for dev in ((xi, yi, left), (xi, yi, right), (1 - xi, yi, zi)):
            pl.semaphore_signal(
                barrier_sem,
                inc=1,
                device_id=dev,
                device_id_type=pl.DeviceIdType.MESH,
            )
        pl.semaphore_wait(barrier_sem, 3)

        def compute_chunk(c, xh):
            partial_ref[pl.ds(c * CHUNK, CHUNK), :] = lax.dot_general(
                x_ref[:, pl.ds(c * CHUNK, CHUNK)],
                dy_ref[:, pl.ds(xh * nh, nh)],
                dimension_numbers=(((0,), (0,)), ((), ())),
                preferred_element_type=jnp.float32,
            )

        def pblock(c, b):
            return partial_ref[pl.ds(c * CHUNK + b * BLK, BLK), :]

        def rfwd(c, b):
            return pltpu.make_async_remote_copy(
                src_ref=rrecv.at[c, b],
                dst_ref=rrecv.at[c, b],
                send_sem=rssem.at[c, b],
                recv_sem=rrsem.at[c, b],
                device_id=(xi, yi, right),
                device_id_type=pl.DeviceIdType.MESH,
            )

        def lfwd(c, b):
            return pltpu.make_async_remote_copy(
                src_ref=lrecv.at[c, b],
                dst_ref=lrecv.at[c, b],
                send_sem=lssem.at[c, b],
                recv_sem=lrsem.at[c, b],
                device_id=(xi, yi, left),
                device_id_type=pl.DeviceIdType.MESH,
            )

        def xswap(b):
            return pltpu.make_async_remote_copy(
                src_ref=xsend.at[b],
                dst_ref=xrecv.at[b],
                send_sem=xssem.at[b],
                recv_sem=xrsem.at[b],
                device_id=(1 - xi, yi, zi),
                device_id_type=pl.DeviceIdType.MESH,
            )

        for xh in (0, 1):

            @pl.when((xi == xh) & (zi <= 1))
            def _(xh=xh):
                for c in (3, 2, 1):
                    compute_chunk(c, xh)

                    @pl.when(zi == 0)
                    def _(c=c):
                        for b in range(NBLK):
                            rrecv[c, b, :, :] = pblock(c, b).astype(WIRE_DTYPE)
                            rfwd(c, b).start()

                compute_chunk(0, xh)

            @pl.when((xi == xh) & (zi >= 2))
            def _(xh=xh):
                for c in (0, 1, 2):
                    compute_chunk(c, xh)

                    @pl.when(zi == 3)
                    def _(c=c):
                        for b in range(NBLK):
                            lrecv[c, b, :, :] = pblock(c, b).astype(WIRE_DTYPE)
                            lfwd(c, b).start()

                compute_chunk(3, xh)

        for s in (1, 2, 3):
            for rc in (1, 2, 3):
                z_m = rc + s - 3
                if 1 <= z_m < rc:

                    @pl.when(zi == z_m)
                    def _(rc=rc, z_m=z_m):
                        for b in range(NBLK):
                            rfwd(rc, b).wait_recv()
                            rrecv[rc, b, :, :] = (
                                rrecv[rc, b, :, :].astype(jnp.float32)
                                + pblock(rc, b)
                            ).astype(WIRE_DTYPE)
                            rfwd(rc, b).start()

            for lc in (0, 1, 2):
                z_m = lc + 3 - s
                if lc < z_m <= 2:

                    @pl.when(zi == z_m)
                    def _(lc=lc, z_m=z_m):
                        for b in range(NBLK):
                            lfwd(lc, b).wait_recv()
                            lrecv[lc, b, :, :] = (
                                lrecv[lc, b, :, :].astype(jnp.float32)
                                + pblock(lc, b)
                            ).astype(WIRE_DTYPE)
                            lfwd(lc, b).start()

        for z_s in range(N_RING):

            @pl.when(zi == z_s)
            def _(z_s=z_s):
                for b in range(NBLK):
                    if z_s >= 1:
                        rfwd(z_s, b).wait_recv()
                    if z_s <= 2:
                        lfwd(z_s, b).wait_recv()
                    val = pblock(z_s, b)
                    if z_s >= 1:
                        val = val + rrecv[z_s, b, :, :].astype(jnp.float32)
                    if z_s <= 2:
                        val = val + lrecv[z_s, b, :, :].astype(jnp.float32)
                    xsend[b, :, :] = val.astype(WIRE_DTYPE)
                    xswap(b).start()
                    for xh in (0, 1):

                        @pl.when(xi == xh)
                        def _(xh=xh, b=b, val=val):
                            out_ref[pl.ds(b * BLK, BLK), pl.ds(xh * nh, nh)] = val

        for b in range(NBLK):
            xswap(b).wait_recv()
            for xh in (0, 1):

                @pl.when(xi == xh)
                def _(xh=xh, b=b):
                    out_ref[
                        pl.ds(b * BLK, BLK), pl.ds((1 - xh) * nh, nh)
                    ] = xrecv[b, :, :].astype(jnp.float32)

        for b in range(NBLK):
            xswap(b).wait_send()
        for z_s in range(N_RING):

            @pl.when(zi == z_s)
            def _(z_s=z_s):
                for rc in range(z_s + 1, N_RING):
                    for b in range(NBLK):
                        rfwd(rc, b).wait_send()
                for lc in range(0, z_s):
                    for b in range(NBLK):
                        lfwd(lc, b).wait_send()

    return pl.pallas_call(
        body,
        out_shape=jax.ShapeDtypeStruct((CHUNK, n), jnp.float32),
        in_specs=[
            pl.BlockSpec(memory_space=pltpu.VMEM),
            pl.BlockSpec(memory_space=pltpu.VMEM),
        ],
        out_specs=pl.BlockSpec(memory_space=pltpu.VMEM),
        scratch_shapes=[
            pltpu.VMEM((m, nh), jnp.float32),
            pltpu.VMEM((N_RING, NBLK, BLK, nh), WIRE_DTYPE),
            pltpu.VMEM((N_RING, NBLK, BLK, nh), WIRE_DTYPE),
            pltpu.VMEM((NBLK, BLK, nh), WIRE_DTYPE),
            pltpu.VMEM((NBLK, BLK, nh), WIRE_DTYPE),
            pltpu.SemaphoreType.DMA((N_RING, NBLK)),
            pltpu.SemaphoreType.DMA((N_RING, NBLK)),
            pltpu.SemaphoreType.DMA((N_RING, NBLK)),
            pltpu.SemaphoreType.DMA((N_RING, NBLK)),
            pltpu.SemaphoreType.DMA((NBLK,)),
            pltpu.SemaphoreType.DMA((NBLK,)),
        ],
        compiler_params=pltpu.CompilerParams(collective_id=0),
    )(x, dy)
